# baseline (speedup 1.0000x reference)
import functools

import jax
import jax.numpy as jnp
from jax import lax
from jax.experimental import pallas as pl
from jax.experimental.pallas import tpu as pltpu
from jax.experimental.pallas import tpu_sc as plsc

H = 2048
I = 1408
E = 7
NK = 2
N = 4096
TM = 256
INB = 128
NI = I // INB
CR = N * NK + E * TM
P = ((CR + N + 511) // 512) * 512
NBLK = P // TM
TB = 512
NW = 32
RPW = P // NW
TPW = N // NW
CH = 16
CH2 = 16


def _router_body(x_ref, wc_ref, wg_ref, sb_ref, idx_ref, w_ref):
    xb = x_ref[...]
    c = jnp.dot(xb, wc_ref[...], preferred_element_type=jnp.float32)
    g = jnp.dot(xb, wg_ref[...], preferred_element_type=jnp.float32)
    s = jnp.abs(c * (g * jax.nn.sigmoid(g)))
    lanes = lax.broadcasted_iota(jnp.int32, (TB, 128), 1)
    valid = lanes < E
    neg = jnp.float32(-jnp.inf)
    s = jnp.where(valid, s, neg)
    mx = jnp.max(s, axis=1, keepdims=True)
    ex = jnp.where(valid, jnp.exp(s - mx), 0.0)
    sm = ex / jnp.sum(ex, axis=1, keepdims=True)
    scale_row = sb_ref[0:1, :]
    bias_row = sb_ref[1:2, :]
    biased = jnp.where(valid, sm + bias_row, neg)
    v0 = jnp.max(biased, axis=1, keepdims=True)
    i0 = jnp.min(jnp.where(biased == v0, lanes, 128), axis=1, keepdims=True)
    b2 = jnp.where(lanes == i0, neg, biased)
    v1 = jnp.max(b2, axis=1, keepdims=True)
    i1 = jnp.min(jnp.where(b2 == v1, lanes, 128), axis=1, keepdims=True)
    scaled = 1.0 + sm * scale_row
    w0 = jnp.sum(jnp.where(lanes == i0, scaled, 0.0), axis=1, keepdims=True)
    w1 = jnp.sum(jnp.where(lanes == i1, scaled, 0.0), axis=1, keepdims=True)
    idx_ref[...] = jnp.concatenate([i0, i1], axis=1)
    w_ref[...] = jnp.concatenate([w0, w1], axis=1)


def _router(xf, wc_pad, wg_pad, sb):
    return pl.pallas_call(
        _router_body,
        grid=(N // TB,),
        in_specs=[
            pl.BlockSpec((TB, H), lambda t: (t, 0)),
            pl.BlockSpec((H, 128), lambda t: (0, 0)),
            pl.BlockSpec((H, 128), lambda t: (0, 0)),
            pl.BlockSpec((8, 128), lambda t: (0, 0)),
        ],
        out_specs=[
            pl.BlockSpec((TB, 2), lambda t: (t, 0)),
            pl.BlockSpec((TB, 2), lambda t: (t, 0)),
        ],
        out_shape=[
            jax.ShapeDtypeStruct((N, 2), jnp.int32),
            jax.ShapeDtypeStruct((N, 2), jnp.float32),
        ],
    )(xf, wc_pad, wg_pad, sb)


def _meta_body(idx_ref, pos_ref, gid_ref):
    MB = 512
    rows = lax.broadcasted_iota(jnp.int32, (MB, MB), 0)
    cols = lax.broadcasted_iota(jnp.int32, (MB, MB), 1)
    tril = jnp.where(rows >= cols, 1.0, 0.0).astype(jnp.float32)
    lanes = lax.broadcasted_iota(jnp.int32, (MB, 128), 1)

    cnt = jnp.zeros((1, 128), jnp.float32)
    for k in range(NK):
        for blk in range(N // MB):
            e = idx_ref[pl.ds(blk * MB, MB), k:k + 1]
            oh = (e == lanes).astype(jnp.float32)
            cnt = cnt + jnp.sum(oh, axis=0, keepdims=True)
    cnti = cnt.astype(jnp.int32)
    padded = ((cnti + TM - 1) // TM) * TM
    r128 = lax.broadcasted_iota(jnp.int32, (128, 128), 0)
    c128 = lax.broadcasted_iota(jnp.int32, (128, 128), 1)
    sut = jnp.where(r128 < c128, 1.0, 0.0).astype(jnp.float32)
    off = jnp.dot(padded.astype(jnp.float32), sut,
                  preferred_element_type=jnp.float32)

    run = jnp.zeros((1, 128), jnp.float32)
    for k in range(NK):
        for blk in range(N // MB):
            e = idx_ref[pl.ds(blk * MB, MB), k:k + 1]
            oh = (e == lanes).astype(jnp.float32)
            csum = jnp.dot(tril, oh, preferred_element_type=jnp.float32) + run
            rank = jnp.sum(csum * oh, axis=1, keepdims=True) - 1.0
            base = jnp.sum(oh * off, axis=1, keepdims=True)
            pos_ref[pl.ds(blk * MB, MB), k:k + 1] = (rank + base).astype(jnp.int32)
            run = csum[MB - 1:MB, :]

    offi = off.astype(jnp.int32)
    mcol = lax.broadcasted_iota(jnp.int32, (64, 1), 0) * TM
    lanes64 = lax.broadcasted_iota(jnp.int32, (64, 128), 1)
    lm = jnp.logical_and(lanes64 >= 1, lanes64 < E)
    a = jnp.logical_and(mcol >= offi, lm).astype(jnp.int32)
    gid = jnp.sum(a, axis=1, keepdims=True) + (mcol >= CR).astype(jnp.int32)
    gid_ref[...] = gid


def _metadata(idx):
    return pl.pallas_call(
        _meta_body,
        grid=(1,),
        in_specs=[pl.BlockSpec((N, 2), lambda _: (0, 0))],
        out_specs=[
            pl.BlockSpec((N, 2), lambda _: (0, 0)),
            pl.BlockSpec((64, 1), lambda _: (0, 0)),
        ],
        out_shape=[
            jax.ShapeDtypeStruct((N, 2), jnp.int32),
            jax.ShapeDtypeStruct((64, 1), jnp.int32),
        ],
    )(idx)


def _dispatch_body(x_hbm, pos_hbm, wp_hbm, xs_hbm, wsl_hbm,
                   tokv, wv, posv, wpv, rowsv, sem):
    core = lax.axis_index("c")
    sub = lax.axis_index("s")
    wid = sub * 2 + core
    pltpu.sync_copy(pos_hbm, posv)
    pltpu.sync_copy(wp_hbm, wpv)

    lane = lax.broadcasted_iota(jnp.int32, (16,), 0)

    def init_body(i, _):
        base = i * 16 + lane
        tokv[pl.ds(i * 16, 16)] = jnp.minimum(jnp.maximum(base - CR, 0), N - 1)
        wv[pl.ds(i * 16, 16)] = jnp.where(base >= CR, 1.0, 0.0).astype(jnp.float32)
        return 0

    lax.fori_loop(0, P // 16, init_body, 0)

    def sc_body(i, _):
        pv = posv[pl.ds(i * 16, 16)]
        tv = lax.shift_right_logical(i * 16 + lane, 1)
        wvv = wpv[pl.ds(i * 16, 16)]
        plsc.store_scatter(tokv, [pv], tv)
        plsc.store_scatter(wv, [pv], wvv)
        return 0

    lax.fori_loop(0, (N * NK) // 16, sc_body, 0)

    base = wid * RPW
    pltpu.sync_copy(wv.at[pl.ds(base, RPW)], wsl_hbm.at[pl.ds(base, RPW)])

    def g_body(c, _):
        start = base + c * CH
        cp = pltpu.async_copy(x_hbm.at[tokv.at[pl.ds(start, CH)]], rowsv, sem)
        cp.wait()
        pltpu.sync_copy(rowsv, xs_hbm.at[pl.ds(start, CH)])
        return 0

    lax.fori_loop(0, RPW // CH, g_body, 0)


def _dispatch(xf, pos2, wp2):
    mesh = plsc.VectorSubcoreMesh(core_axis_name="c", subcore_axis_name="s")
    f = pl.kernel(
        _dispatch_body,
        out_type=[
            jax.ShapeDtypeStruct((P, H), jnp.float32),
            jax.ShapeDtypeStruct((P,), jnp.float32),
        ],
        mesh=mesh,
        scratch_types=[
            pltpu.VMEM((P,), jnp.int32),
            pltpu.VMEM((P,), jnp.float32),
            pltpu.VMEM((N * NK,), jnp.int32),
            pltpu.VMEM((N * NK,), jnp.float32),
            pltpu.VMEM((CH, H), jnp.float32),
            pltpu.SemaphoreType.DMA,
        ],
        compiler_params=pltpu.CompilerParams(needs_layout_passes=False),
    )
    return f(xf, pos2, wp2)


def _gmm_body(gid_ref, xs_ref, wg_ref, wu_ref, wd_ref, wsl_ref, out_ref):
    n = pl.program_id(1)
    xb = xs_ref[...]
    g = jnp.dot(xb, wg_ref[0], preferred_element_type=jnp.float32)
    u = jnp.dot(xb, wu_ref[0], preferred_element_type=jnp.float32)
    h = g * jax.nn.sigmoid(g) * u
    part = jnp.dot(h, wd_ref[0], preferred_element_type=jnp.float32)
    part = part * wsl_ref[...]

    @pl.when(n == 0)
    def _():
        out_ref[...] = part

    @pl.when(n != 0)
    def _():
        out_ref[...] += part


def _gmm(gid1d, xs, wg_all, wu_all, wd_all, wsl2):
    grid_spec = pltpu.PrefetchScalarGridSpec(
        num_scalar_prefetch=1,
        grid=(NBLK, NI),
        in_specs=[
            pl.BlockSpec((TM, H), lambda m, n, gid: (m, 0)),
            pl.BlockSpec((1, H, INB), lambda m, n, gid: (gid[m], 0, n)),
            pl.BlockSpec((1, H, INB), lambda m, n, gid: (gid[m], 0, n)),
            pl.BlockSpec((1, INB, H), lambda m, n, gid: (gid[m], n, 0)),
            pl.BlockSpec((TM, 1), lambda m, n, gid: (m, 0)),
        ],
        out_specs=pl.BlockSpec((TM, H), lambda m, n, gid: (m, 0)),
    )
    return pl.pallas_call(
        _gmm_body,
        grid_spec=grid_spec,
        out_shape=jax.ShapeDtypeStruct((P, H), jnp.float32),
        compiler_params=pltpu.CompilerParams(
            dimension_semantics=("arbitrary", "arbitrary"),
        ),
    )(gid1d, xs, wg_all, wu_all, wd_all, wsl2)


def _combine_body(eo_hbm, p0_hbm, p1_hbm, out_hbm,
                  i0v, i1v, bufa, bufb, bufc, sema, semb, semc):
    core = lax.axis_index("c")
    sub = lax.axis_index("s")
    wid = sub * 2 + core
    tb = wid * TPW
    pltpu.sync_copy(p0_hbm.at[pl.ds(tb, TPW)], i0v)
    pltpu.sync_copy(p1_hbm.at[pl.ds(tb, TPW)], i1v)

    def chunk(c, _):
        s = tb + c * CH2
        cpa = pltpu.async_copy(eo_hbm.at[i0v.at[pl.ds(c * CH2, CH2)]], bufa, sema)
        cpb = pltpu.async_copy(eo_hbm.at[i1v.at[pl.ds(c * CH2, CH2)]], bufb, semb)
        cpc = pltpu.async_copy(eo_hbm.at[pl.ds(CR + s, CH2)], bufc, semc)
        cpa.wait()
        cpb.wait()
        cpc.wait()

        def row(r, _):
            def seg(cb, _):
                d = pl.ds(cb * 16, 16)
                bufa[r, d] = bufa[r, d] + bufb[r, d] + bufc[r, d]
                return 0
            lax.fori_loop(0, H // 16, seg, 0)
            return 0

        lax.fori_loop(0, CH2, row, 0)
        pltpu.sync_copy(bufa, out_hbm.at[pl.ds(s, CH2)])
        return 0

    lax.fori_loop(0, TPW // CH2, chunk, 0)


def _combine(eo, p0, p1):
    mesh = plsc.VectorSubcoreMesh(core_axis_name="c", subcore_axis_name="s")
    f = pl.kernel(
        _combine_body,
        out_type=jax.ShapeDtypeStruct((N, H), jnp.float32),
        mesh=mesh,
        scratch_types=[
            pltpu.VMEM((TPW,), jnp.int32),
            pltpu.VMEM((TPW,), jnp.int32),
            pltpu.VMEM((CH2, H), jnp.float32),
            pltpu.VMEM((CH2, H), jnp.float32),
            pltpu.VMEM((CH2, H), jnp.float32),
            pltpu.SemaphoreType.DMA,
            pltpu.SemaphoreType.DMA,
            pltpu.SemaphoreType.DMA,
        ],
        compiler_params=pltpu.CompilerParams(needs_layout_passes=False),
    )
    return f(eo, p0, p1)


def kernel(x, router_gate_W, router_cls_W, extra_scale, extra_bias,
           expert_Wg, expert_Wu, expert_Wd, shared_Wg, shared_Wu, shared_Wd):
    xf = x.reshape(-1, H)
    wc_pad = jnp.zeros((H, 128), jnp.float32).at[:, :E].set(router_cls_W)
    wg_pad = jnp.zeros((H, 128), jnp.float32).at[:, :E].set(router_gate_W)
    sb = (jnp.zeros((8, 128), jnp.float32)
          .at[0, :E].set(extra_scale)
          .at[1, :E].set(extra_bias))
    idx, w = _router(xf, wc_pad, wg_pad, sb)
    pos, gid = _metadata(idx)
    gid1d = gid[:, 0]
    pos2 = pos.reshape(-1)
    wp2 = w.reshape(-1)
    xs, wsl = _dispatch(xf, pos2, wp2)
    wg_all = jnp.concatenate([expert_Wg, shared_Wg[None]], axis=0)
    wu_all = jnp.concatenate([expert_Wu, shared_Wu[None]], axis=0)
    wd_all = jnp.concatenate([expert_Wd, shared_Wd[None]], axis=0)
    eo = _gmm(gid1d, xs, wg_all, wu_all, wd_all, wsl.reshape(P, 1))
    out = _combine(eo, pos[:, 0], pos[:, 1])
    return out.reshape(x.shape)


# speedup vs baseline: 1.0682x; 1.0682x over previous
import functools

import jax
import jax.numpy as jnp
from jax import lax
from jax.experimental import pallas as pl
from jax.experimental.pallas import tpu as pltpu
from jax.experimental.pallas import tpu_sc as plsc

H = 2048
I = 1408
E = 7
NK = 2
N = 4096
TM = 256
INB = 128
NI = I // INB
CR = N * NK + E * TM
P = ((CR + N + 511) // 512) * 512
NBLK = P // TM
TB = 512
NW = 32
RPW = P // NW
TPW = N // NW
CH = 16
CH2 = 16


def _router_body(x_ref, wc_ref, wg_ref, sb_ref, idx_ref, w_ref):
    xb = x_ref[...]
    c = jnp.dot(xb, wc_ref[...], preferred_element_type=jnp.float32)
    g = jnp.dot(xb, wg_ref[...], preferred_element_type=jnp.float32)
    s = jnp.abs(c * (g * jax.nn.sigmoid(g)))
    lanes = lax.broadcasted_iota(jnp.int32, (TB, 128), 1)
    valid = lanes < E
    neg = jnp.float32(-jnp.inf)
    s = jnp.where(valid, s, neg)
    mx = jnp.max(s, axis=1, keepdims=True)
    ex = jnp.where(valid, jnp.exp(s - mx), 0.0)
    sm = ex / jnp.sum(ex, axis=1, keepdims=True)
    scale_row = sb_ref[0:1, :]
    bias_row = sb_ref[1:2, :]
    biased = jnp.where(valid, sm + bias_row, neg)
    v0 = jnp.max(biased, axis=1, keepdims=True)
    i0 = jnp.min(jnp.where(biased == v0, lanes, 128), axis=1, keepdims=True)
    b2 = jnp.where(lanes == i0, neg, biased)
    v1 = jnp.max(b2, axis=1, keepdims=True)
    i1 = jnp.min(jnp.where(b2 == v1, lanes, 128), axis=1, keepdims=True)
    scaled = 1.0 + sm * scale_row
    w0 = jnp.sum(jnp.where(lanes == i0, scaled, 0.0), axis=1, keepdims=True)
    w1 = jnp.sum(jnp.where(lanes == i1, scaled, 0.0), axis=1, keepdims=True)
    idx_ref[...] = jnp.concatenate([i0, i1], axis=1)
    w_ref[...] = jnp.concatenate([w0, w1], axis=1)


def _router(xf, wc_pad, wg_pad, sb):
    return pl.pallas_call(
        _router_body,
        grid=(N // TB,),
        in_specs=[
            pl.BlockSpec((TB, H), lambda t: (t, 0)),
            pl.BlockSpec((H, 128), lambda t: (0, 0)),
            pl.BlockSpec((H, 128), lambda t: (0, 0)),
            pl.BlockSpec((8, 128), lambda t: (0, 0)),
        ],
        out_specs=[
            pl.BlockSpec((TB, 2), lambda t: (t, 0)),
            pl.BlockSpec((TB, 2), lambda t: (t, 0)),
        ],
        out_shape=[
            jax.ShapeDtypeStruct((N, 2), jnp.int32),
            jax.ShapeDtypeStruct((N, 2), jnp.float32),
        ],
    )(xf, wc_pad, wg_pad, sb)


def _meta_body(idx_ref, pos_ref, gid_ref):
    MB = 512
    rows = lax.broadcasted_iota(jnp.int32, (MB, MB), 0)
    cols = lax.broadcasted_iota(jnp.int32, (MB, MB), 1)
    tril = jnp.where(rows >= cols, 1.0, 0.0).astype(jnp.float32)
    lanes = lax.broadcasted_iota(jnp.int32, (MB, 128), 1)

    cnt = jnp.zeros((1, 128), jnp.float32)
    for k in range(NK):
        for blk in range(N // MB):
            e = idx_ref[pl.ds(blk * MB, MB), k:k + 1]
            oh = (e == lanes).astype(jnp.float32)
            cnt = cnt + jnp.sum(oh, axis=0, keepdims=True)
    cnti = cnt.astype(jnp.int32)
    padded = ((cnti + TM - 1) // TM) * TM
    r128 = lax.broadcasted_iota(jnp.int32, (128, 128), 0)
    c128 = lax.broadcasted_iota(jnp.int32, (128, 128), 1)
    sut = jnp.where(r128 < c128, 1.0, 0.0).astype(jnp.float32)
    off = jnp.dot(padded.astype(jnp.float32), sut,
                  preferred_element_type=jnp.float32)

    run = jnp.zeros((1, 128), jnp.float32)
    for k in range(NK):
        for blk in range(N // MB):
            e = idx_ref[pl.ds(blk * MB, MB), k:k + 1]
            oh = (e == lanes).astype(jnp.float32)
            csum = jnp.dot(tril, oh, preferred_element_type=jnp.float32) + run
            rank = jnp.sum(csum * oh, axis=1, keepdims=True) - 1.0
            base = jnp.sum(oh * off, axis=1, keepdims=True)
            pos_ref[pl.ds(blk * MB, MB), k:k + 1] = (rank + base).astype(jnp.int32)
            run = csum[MB - 1:MB, :]

    offi = off.astype(jnp.int32)
    mcol = lax.broadcasted_iota(jnp.int32, (64, 1), 0) * TM
    lanes64 = lax.broadcasted_iota(jnp.int32, (64, 128), 1)
    lm = jnp.logical_and(lanes64 >= 1, lanes64 < E)
    a = jnp.logical_and(mcol >= offi, lm).astype(jnp.int32)
    gid = jnp.sum(a, axis=1, keepdims=True) + (mcol >= CR).astype(jnp.int32)
    gid_ref[...] = gid


def _metadata(idx):
    return pl.pallas_call(
        _meta_body,
        grid=(1,),
        in_specs=[pl.BlockSpec((N, 2), lambda _: (0, 0))],
        out_specs=[
            pl.BlockSpec((N, 2), lambda _: (0, 0)),
            pl.BlockSpec((64, 1), lambda _: (0, 0)),
        ],
        out_shape=[
            jax.ShapeDtypeStruct((N, 2), jnp.int32),
            jax.ShapeDtypeStruct((64, 1), jnp.int32),
        ],
    )(idx)


def _dispatch_body(x_hbm, pos_hbm, wp_hbm, xs_hbm, wsl_hbm,
                   tokv, wv, posv, wpv, rowsv, sem):
    core = lax.axis_index("c")
    sub = lax.axis_index("s")
    wid = sub * 2 + core
    pltpu.sync_copy(pos_hbm, posv)
    pltpu.sync_copy(wp_hbm, wpv)

    lane = lax.broadcasted_iota(jnp.int32, (16,), 0)

    def init_body(i, _):
        base = i * 16 + lane
        tokv[pl.ds(i * 16, 16)] = jnp.minimum(jnp.maximum(base - CR, 0), N - 1)
        wv[pl.ds(i * 16, 16)] = jnp.where(base >= CR, 1.0, 0.0).astype(jnp.float32)
        return 0

    lax.fori_loop(0, P // 16, init_body, 0)

    def sc_body(i, _):
        pv = posv[pl.ds(i * 16, 16)]
        tv = lax.shift_right_logical(i * 16 + lane, 1)
        wvv = wpv[pl.ds(i * 16, 16)]
        plsc.store_scatter(tokv, [pv], tv)
        plsc.store_scatter(wv, [pv], wvv)
        return 0

    lax.fori_loop(0, (N * NK) // 16, sc_body, 0)

    base = wid * RPW
    pltpu.sync_copy(wv.at[pl.ds(base, RPW)], wsl_hbm.at[pl.ds(base, RPW)])

    def g_body(c, _):
        start = base + c * CH
        cp = pltpu.async_copy(x_hbm.at[tokv.at[pl.ds(start, CH)]], rowsv, sem)
        cp.wait()
        pltpu.sync_copy(rowsv, xs_hbm.at[pl.ds(start, CH)])
        return 0

    lax.fori_loop(0, RPW // CH, g_body, 0)


def _dispatch(xf, pos2, wp2):
    mesh = plsc.VectorSubcoreMesh(core_axis_name="c", subcore_axis_name="s")
    f = pl.kernel(
        _dispatch_body,
        out_type=[
            jax.ShapeDtypeStruct((P, H), jnp.float32),
            jax.ShapeDtypeStruct((P,), jnp.float32),
        ],
        mesh=mesh,
        scratch_types=[
            pltpu.VMEM((P,), jnp.int32),
            pltpu.VMEM((P,), jnp.float32),
            pltpu.VMEM((N * NK,), jnp.int32),
            pltpu.VMEM((N * NK,), jnp.float32),
            pltpu.VMEM((CH, H), jnp.float32),
            pltpu.SemaphoreType.DMA,
        ],
        compiler_params=pltpu.CompilerParams(needs_layout_passes=False),
    )
    return f(xf, pos2, wp2)


def _gmm_body(gid_ref, xs_ref, wg_ref, wu_ref, wd_ref, wsl_ref, out_ref):
    n = pl.program_id(1)
    xb = xs_ref[...].astype(jnp.bfloat16)
    g = jnp.dot(xb, wg_ref[0], preferred_element_type=jnp.float32)
    u = jnp.dot(xb, wu_ref[0], preferred_element_type=jnp.float32)
    h = (g * jax.nn.sigmoid(g) * u).astype(jnp.bfloat16)
    part = jnp.dot(h, wd_ref[0], preferred_element_type=jnp.float32)
    part = part * wsl_ref[...]

    @pl.when(n == 0)
    def _():
        out_ref[...] = part

    @pl.when(n != 0)
    def _():
        out_ref[...] += part


def _gmm(gid1d, xs, wg_all, wu_all, wd_all, wsl2):
    grid_spec = pltpu.PrefetchScalarGridSpec(
        num_scalar_prefetch=1,
        grid=(NBLK, NI),
        in_specs=[
            pl.BlockSpec((TM, H), lambda m, n, gid: (m, 0)),
            pl.BlockSpec((1, H, INB), lambda m, n, gid: (gid[m], 0, n)),
            pl.BlockSpec((1, H, INB), lambda m, n, gid: (gid[m], 0, n)),
            pl.BlockSpec((1, INB, H), lambda m, n, gid: (gid[m], n, 0)),
            pl.BlockSpec((TM, 1), lambda m, n, gid: (m, 0)),
        ],
        out_specs=pl.BlockSpec((TM, H), lambda m, n, gid: (m, 0)),
    )
    return pl.pallas_call(
        _gmm_body,
        grid_spec=grid_spec,
        out_shape=jax.ShapeDtypeStruct((P, H), jnp.float32),
        compiler_params=pltpu.CompilerParams(
            dimension_semantics=("arbitrary", "arbitrary"),
        ),
    )(gid1d, xs, wg_all, wu_all, wd_all, wsl2)


def _combine_body(eo_hbm, p0_hbm, p1_hbm, out_hbm,
                  i0v, i1v, bufa, bufb, bufc, sema, semb, semc):
    core = lax.axis_index("c")
    sub = lax.axis_index("s")
    wid = sub * 2 + core
    tb = wid * TPW
    pltpu.sync_copy(p0_hbm.at[pl.ds(tb, TPW)], i0v)
    pltpu.sync_copy(p1_hbm.at[pl.ds(tb, TPW)], i1v)

    def chunk(c, _):
        s = tb + c * CH2
        cpa = pltpu.async_copy(eo_hbm.at[i0v.at[pl.ds(c * CH2, CH2)]], bufa, sema)
        cpb = pltpu.async_copy(eo_hbm.at[i1v.at[pl.ds(c * CH2, CH2)]], bufb, semb)
        cpc = pltpu.async_copy(eo_hbm.at[pl.ds(CR + s, CH2)], bufc, semc)
        cpa.wait()
        cpb.wait()
        cpc.wait()

        def row(r, _):
            def seg(cb, _):
                d = pl.ds(cb * 16, 16)
                bufa[r, d] = bufa[r, d] + bufb[r, d] + bufc[r, d]
                return 0
            lax.fori_loop(0, H // 16, seg, 0)
            return 0

        lax.fori_loop(0, CH2, row, 0)
        pltpu.sync_copy(bufa, out_hbm.at[pl.ds(s, CH2)])
        return 0

    lax.fori_loop(0, TPW // CH2, chunk, 0)


def _combine(eo, p0, p1):
    mesh = plsc.VectorSubcoreMesh(core_axis_name="c", subcore_axis_name="s")
    f = pl.kernel(
        _combine_body,
        out_type=jax.ShapeDtypeStruct((N, H), jnp.float32),
        mesh=mesh,
        scratch_types=[
            pltpu.VMEM((TPW,), jnp.int32),
            pltpu.VMEM((TPW,), jnp.int32),
            pltpu.VMEM((CH2, H), jnp.float32),
            pltpu.VMEM((CH2, H), jnp.float32),
            pltpu.VMEM((CH2, H), jnp.float32),
            pltpu.SemaphoreType.DMA,
            pltpu.SemaphoreType.DMA,
            pltpu.SemaphoreType.DMA,
        ],
        compiler_params=pltpu.CompilerParams(needs_layout_passes=False),
    )
    return f(eo, p0, p1)


def kernel(x, router_gate_W, router_cls_W, extra_scale, extra_bias,
           expert_Wg, expert_Wu, expert_Wd, shared_Wg, shared_Wu, shared_Wd):
    xf = x.reshape(-1, H)
    wc_pad = jnp.zeros((H, 128), jnp.float32).at[:, :E].set(router_cls_W)
    wg_pad = jnp.zeros((H, 128), jnp.float32).at[:, :E].set(router_gate_W)
    sb = (jnp.zeros((8, 128), jnp.float32)
          .at[0, :E].set(extra_scale)
          .at[1, :E].set(extra_bias))
    idx, w = _router(xf, wc_pad, wg_pad, sb)
    pos, gid = _metadata(idx)
    gid1d = gid[:, 0]
    pos2 = pos.reshape(-1)
    wp2 = w.reshape(-1)
    xs, wsl = _dispatch(xf, pos2, wp2)
    wg_all = jnp.concatenate([expert_Wg, shared_Wg[None]], axis=0).astype(jnp.bfloat16)
    wu_all = jnp.concatenate([expert_Wu, shared_Wu[None]], axis=0).astype(jnp.bfloat16)
    wd_all = jnp.concatenate([expert_Wd, shared_Wd[None]], axis=0).astype(jnp.bfloat16)
    eo = _gmm(gid1d, xs, wg_all, wu_all, wd_all, wsl.reshape(P, 1))
    out = _combine(eo, pos[:, 0], pos[:, 1])
    return out.reshape(x.shape)


# speedup vs baseline: 1.7495x; 1.6378x over previous
import functools

import jax
import jax.numpy as jnp
from jax import lax
from jax.experimental import pallas as pl
from jax.experimental.pallas import tpu as pltpu
from jax.experimental.pallas import tpu_sc as plsc

H = 2048
I = 1408
E = 7
NK = 2
N = 4096
TM = 256
INB = 128
NI = I // INB
CR = N * NK + E * TM
P = ((CR + N + 511) // 512) * 512
NBLK = P // TM
TB = 512
NW = 32
RPW = P // NW
TPW = N // NW
CH = 16
CH2 = 16


def _router_body(x_ref, wc_ref, wg_ref, sb_ref, idx_ref, w_ref):
    xb = x_ref[...]
    c = jnp.dot(xb, wc_ref[...], preferred_element_type=jnp.float32)
    g = jnp.dot(xb, wg_ref[...], preferred_element_type=jnp.float32)
    s = jnp.abs(c * (g * jax.nn.sigmoid(g)))
    lanes = lax.broadcasted_iota(jnp.int32, (TB, 128), 1)
    valid = lanes < E
    neg = jnp.float32(-jnp.inf)
    s = jnp.where(valid, s, neg)
    mx = jnp.max(s, axis=1, keepdims=True)
    ex = jnp.where(valid, jnp.exp(s - mx), 0.0)
    sm = ex / jnp.sum(ex, axis=1, keepdims=True)
    scale_row = sb_ref[0:1, :]
    bias_row = sb_ref[1:2, :]
    biased = jnp.where(valid, sm + bias_row, neg)
    v0 = jnp.max(biased, axis=1, keepdims=True)
    i0 = jnp.min(jnp.where(biased == v0, lanes, 128), axis=1, keepdims=True)
    b2 = jnp.where(lanes == i0, neg, biased)
    v1 = jnp.max(b2, axis=1, keepdims=True)
    i1 = jnp.min(jnp.where(b2 == v1, lanes, 128), axis=1, keepdims=True)
    scaled = 1.0 + sm * scale_row
    w0 = jnp.sum(jnp.where(lanes == i0, scaled, 0.0), axis=1, keepdims=True)
    w1 = jnp.sum(jnp.where(lanes == i1, scaled, 0.0), axis=1, keepdims=True)
    idx_ref[...] = jnp.concatenate([i0, i1], axis=1)
    w_ref[...] = jnp.concatenate([w0, w1], axis=1)


def _router(xf, wc_pad, wg_pad, sb):
    return pl.pallas_call(
        _router_body,
        grid=(N // TB,),
        in_specs=[
            pl.BlockSpec((TB, H), lambda t: (t, 0)),
            pl.BlockSpec((H, 128), lambda t: (0, 0)),
            pl.BlockSpec((H, 128), lambda t: (0, 0)),
            pl.BlockSpec((8, 128), lambda t: (0, 0)),
        ],
        out_specs=[
            pl.BlockSpec((TB, 2), lambda t: (t, 0)),
            pl.BlockSpec((TB, 2), lambda t: (t, 0)),
        ],
        out_shape=[
            jax.ShapeDtypeStruct((N, 2), jnp.int32),
            jax.ShapeDtypeStruct((N, 2), jnp.float32),
        ],
    )(xf, wc_pad, wg_pad, sb)


def _meta_body(idx_ref, pos_ref, gid_ref):
    MB = 512
    rows = lax.broadcasted_iota(jnp.int32, (MB, MB), 0)
    cols = lax.broadcasted_iota(jnp.int32, (MB, MB), 1)
    tril = jnp.where(rows >= cols, 1.0, 0.0).astype(jnp.float32)
    lanes = lax.broadcasted_iota(jnp.int32, (MB, 128), 1)

    cnt = jnp.zeros((1, 128), jnp.float32)
    for k in range(NK):
        for blk in range(N // MB):
            e = idx_ref[pl.ds(blk * MB, MB), k:k + 1]
            oh = (e == lanes).astype(jnp.float32)
            cnt = cnt + jnp.sum(oh, axis=0, keepdims=True)
    cnti = cnt.astype(jnp.int32)
    padded = ((cnti + TM - 1) // TM) * TM
    r128 = lax.broadcasted_iota(jnp.int32, (128, 128), 0)
    c128 = lax.broadcasted_iota(jnp.int32, (128, 128), 1)
    sut = jnp.where(r128 < c128, 1.0, 0.0).astype(jnp.float32)
    off = jnp.dot(padded.astype(jnp.float32), sut,
                  preferred_element_type=jnp.float32)

    run = jnp.zeros((1, 128), jnp.float32)
    for k in range(NK):
        for blk in range(N // MB):
            e = idx_ref[pl.ds(blk * MB, MB), k:k + 1]
            oh = (e == lanes).astype(jnp.float32)
            csum = jnp.dot(tril, oh, preferred_element_type=jnp.float32) + run
            rank = jnp.sum(csum * oh, axis=1, keepdims=True) - 1.0
            base = jnp.sum(oh * off, axis=1, keepdims=True)
            pos_ref[pl.ds(blk * MB, MB), k:k + 1] = (rank + base).astype(jnp.int32)
            run = csum[MB - 1:MB, :]

    offi = off.astype(jnp.int32)
    mcol = lax.broadcasted_iota(jnp.int32, (64, 1), 0) * TM
    lanes64 = lax.broadcasted_iota(jnp.int32, (64, 128), 1)
    lm = jnp.logical_and(lanes64 >= 1, lanes64 < E)
    a = jnp.logical_and(mcol >= offi, lm).astype(jnp.int32)
    gid = jnp.sum(a, axis=1, keepdims=True) + (mcol >= CR).astype(jnp.int32)
    gid_ref[...] = gid


def _metadata(idx):
    return pl.pallas_call(
        _meta_body,
        grid=(1,),
        in_specs=[pl.BlockSpec((N, 2), lambda _: (0, 0))],
        out_specs=[
            pl.BlockSpec((N, 2), lambda _: (0, 0)),
            pl.BlockSpec((64, 1), lambda _: (0, 0)),
        ],
        out_shape=[
            jax.ShapeDtypeStruct((N, 2), jnp.int32),
            jax.ShapeDtypeStruct((64, 1), jnp.int32),
        ],
    )(idx)


def _dispatch_body(x_hbm, pos_hbm, wp_hbm, xs_hbm, wsl_hbm,
                   tokv, wv, posv, wpv, rowsv, sem):
    core = lax.axis_index("c")
    sub = lax.axis_index("s")
    wid = sub * 2 + core
    pltpu.sync_copy(pos_hbm, posv)
    pltpu.sync_copy(wp_hbm, wpv)

    lane = lax.broadcasted_iota(jnp.int32, (16,), 0)

    def init_body(i, _):
        base = i * 16 + lane
        tokv[pl.ds(i * 16, 16)] = jnp.minimum(jnp.maximum(base - CR, 0), N - 1)
        wv[pl.ds(i * 16, 16)] = jnp.where(base >= CR, 1.0, 0.0).astype(jnp.float32)
        return 0

    lax.fori_loop(0, P // 16, init_body, 0)

    def sc_body(i, _):
        pv = posv[pl.ds(i * 16, 16)]
        tv = lax.shift_right_logical(i * 16 + lane, 1)
        wvv = wpv[pl.ds(i * 16, 16)]
        plsc.store_scatter(tokv, [pv], tv)
        plsc.store_scatter(wv, [pv], wvv)
        return 0

    lax.fori_loop(0, (N * NK) // 16, sc_body, 0)

    base = wid * RPW
    pltpu.sync_copy(wv.at[pl.ds(base, RPW)], wsl_hbm.at[pl.ds(base, RPW)])

    def g_body(c, _):
        start = base + c * CH
        cp = pltpu.async_copy(x_hbm.at[tokv.at[pl.ds(start, CH)]], rowsv, sem)
        cp.wait()
        pltpu.sync_copy(rowsv, xs_hbm.at[pl.ds(start, CH)])
        return 0

    lax.fori_loop(0, RPW // CH, g_body, 0)


def _dispatch(xf, pos2, wp2):
    mesh = plsc.VectorSubcoreMesh(core_axis_name="c", subcore_axis_name="s")
    f = pl.kernel(
        _dispatch_body,
        out_type=[
            jax.ShapeDtypeStruct((P, H), jnp.float32),
            jax.ShapeDtypeStruct((P,), jnp.float32),
        ],
        mesh=mesh,
        scratch_types=[
            pltpu.VMEM((P,), jnp.int32),
            pltpu.VMEM((P,), jnp.float32),
            pltpu.VMEM((N * NK,), jnp.int32),
            pltpu.VMEM((N * NK,), jnp.float32),
            pltpu.VMEM((CH, H), jnp.float32),
            pltpu.SemaphoreType.DMA,
        ],
        compiler_params=pltpu.CompilerParams(needs_layout_passes=False),
    )
    return f(xf, pos2, wp2)


def _gmm_body(gid_ref, xs_ref, wg_ref, wu_ref, wd_ref, wsl_ref, out_ref):
    xb = xs_ref[...].astype(jnp.bfloat16)
    g = jnp.dot(xb, wg_ref[0], preferred_element_type=jnp.float32)
    u = jnp.dot(xb, wu_ref[0], preferred_element_type=jnp.float32)
    h = (g * jax.nn.sigmoid(g) * u).astype(jnp.bfloat16)
    part = jnp.dot(h, wd_ref[0], preferred_element_type=jnp.float32)
    out_ref[...] = part * wsl_ref[...]


def _gmm(gid1d, xs, wg_all, wu_all, wd_all, wsl2):
    grid_spec = pltpu.PrefetchScalarGridSpec(
        num_scalar_prefetch=1,
        grid=(NBLK,),
        in_specs=[
            pl.BlockSpec((TM, H), lambda m, gid: (m, 0)),
            pl.BlockSpec((1, H, I), lambda m, gid: (gid[m], 0, 0)),
            pl.BlockSpec((1, H, I), lambda m, gid: (gid[m], 0, 0)),
            pl.BlockSpec((1, I, H), lambda m, gid: (gid[m], 0, 0)),
            pl.BlockSpec((TM, 1), lambda m, gid: (m, 0)),
        ],
        out_specs=pl.BlockSpec((TM, H), lambda m, gid: (m, 0)),
    )
    return pl.pallas_call(
        _gmm_body,
        grid_spec=grid_spec,
        out_shape=jax.ShapeDtypeStruct((P, H), jnp.float32),
        compiler_params=pltpu.CompilerParams(
            dimension_semantics=("arbitrary",),
            vmem_limit_bytes=116 * 1024 * 1024,
        ),
    )(gid1d, xs, wg_all, wu_all, wd_all, wsl2)


def _combine_body(eo_hbm, p0_hbm, p1_hbm, out_hbm,
                  i0v, i1v, bufa, bufb, bufc, sema, semb, semc):
    core = lax.axis_index("c")
    sub = lax.axis_index("s")
    wid = sub * 2 + core
    tb = wid * TPW
    pltpu.sync_copy(p0_hbm.at[pl.ds(tb, TPW)], i0v)
    pltpu.sync_copy(p1_hbm.at[pl.ds(tb, TPW)], i1v)

    def chunk(c, _):
        s = tb + c * CH2
        cpa = pltpu.async_copy(eo_hbm.at[i0v.at[pl.ds(c * CH2, CH2)]], bufa, sema)
        cpb = pltpu.async_copy(eo_hbm.at[i1v.at[pl.ds(c * CH2, CH2)]], bufb, semb)
        cpc = pltpu.async_copy(eo_hbm.at[pl.ds(CR + s, CH2)], bufc, semc)
        cpa.wait()
        cpb.wait()
        cpc.wait()

        def row(r, _):
            def seg(cb, _):
                d = pl.ds(cb * 16, 16)
                bufa[r, d] = bufa[r, d] + bufb[r, d] + bufc[r, d]
                return 0
            lax.fori_loop(0, H // 16, seg, 0)
            return 0

        lax.fori_loop(0, CH2, row, 0)
        pltpu.sync_copy(bufa, out_hbm.at[pl.ds(s, CH2)])
        return 0

    lax.fori_loop(0, TPW // CH2, chunk, 0)


def _combine(eo, p0, p1):
    mesh = plsc.VectorSubcoreMesh(core_axis_name="c", subcore_axis_name="s")
    f = pl.kernel(
        _combine_body,
        out_type=jax.ShapeDtypeStruct((N, H), jnp.float32),
        mesh=mesh,
        scratch_types=[
            pltpu.VMEM((TPW,), jnp.int32),
            pltpu.VMEM((TPW,), jnp.int32),
            pltpu.VMEM((CH2, H), jnp.float32),
            pltpu.VMEM((CH2, H), jnp.float32),
            pltpu.VMEM((CH2, H), jnp.float32),
            pltpu.SemaphoreType.DMA,
            pltpu.SemaphoreType.DMA,
            pltpu.SemaphoreType.DMA,
        ],
        compiler_params=pltpu.CompilerParams(needs_layout_passes=False),
    )
    return f(eo, p0, p1)


def kernel(x, router_gate_W, router_cls_W, extra_scale, extra_bias,
           expert_Wg, expert_Wu, expert_Wd, shared_Wg, shared_Wu, shared_Wd):
    xf = x.reshape(-1, H)
    wc_pad = jnp.zeros((H, 128), jnp.float32).at[:, :E].set(router_cls_W)
    wg_pad = jnp.zeros((H, 128), jnp.float32).at[:, :E].set(router_gate_W)
    sb = (jnp.zeros((8, 128), jnp.float32)
          .at[0, :E].set(extra_scale)
          .at[1, :E].set(extra_bias))
    idx, w = _router(xf, wc_pad, wg_pad, sb)
    pos, gid = _metadata(idx)
    gid1d = gid[:, 0]
    pos2 = pos.reshape(-1)
    wp2 = w.reshape(-1)
    xs, wsl = _dispatch(xf, pos2, wp2)
    wg_all = jnp.concatenate([expert_Wg, shared_Wg[None]], axis=0).astype(jnp.bfloat16)
    wu_all = jnp.concatenate([expert_Wu, shared_Wu[None]], axis=0).astype(jnp.bfloat16)
    wd_all = jnp.concatenate([expert_Wd, shared_Wd[None]], axis=0).astype(jnp.bfloat16)
    eo = _gmm(gid1d, xs, wg_all, wu_all, wd_all, wsl.reshape(P, 1))
    out = _combine(eo, pos[:, 0], pos[:, 1])
    return out.reshape(x.shape)


# speedup vs baseline: 1.7686x; 1.0109x over previous
import functools

import jax
import jax.numpy as jnp
from jax import lax
from jax.experimental import pallas as pl
from jax.experimental.pallas import tpu as pltpu
from jax.experimental.pallas import tpu_sc as plsc

H = 2048
I = 1408
E = 7
NK = 2
N = 4096
TM = 256
INB = 128
NI = I // INB
CR = 10240
P = CR + N
NBLK = P // TM
MR = CR // TM
TB = 512
NW = 32
RR = CR // NW
TPW = N // NW
CH = 16
NCH = RR // CH
CH2 = 16


def _router_body(x_ref, wc_ref, wg_ref, sb_ref, idx_ref, w_ref):
    xb = x_ref[...]
    c = jnp.dot(xb, wc_ref[...], preferred_element_type=jnp.float32)
    g = jnp.dot(xb, wg_ref[...], preferred_element_type=jnp.float32)
    s = jnp.abs(c * (g * jax.nn.sigmoid(g)))
    lanes = lax.broadcasted_iota(jnp.int32, (TB, 128), 1)
    valid = lanes < E
    neg = jnp.float32(-jnp.inf)
    s = jnp.where(valid, s, neg)
    mx = jnp.max(s, axis=1, keepdims=True)
    ex = jnp.where(valid, jnp.exp(s - mx), 0.0)
    sm = ex / jnp.sum(ex, axis=1, keepdims=True)
    scale_row = sb_ref[0:1, :]
    bias_row = sb_ref[1:2, :]
    biased = jnp.where(valid, sm + bias_row, neg)
    v0 = jnp.max(biased, axis=1, keepdims=True)
    i0 = jnp.min(jnp.where(biased == v0, lanes, 128), axis=1, keepdims=True)
    b2 = jnp.where(lanes == i0, neg, biased)
    v1 = jnp.max(b2, axis=1, keepdims=True)
    i1 = jnp.min(jnp.where(b2 == v1, lanes, 128), axis=1, keepdims=True)
    scaled = 1.0 + sm * scale_row
    w0 = jnp.sum(jnp.where(lanes == i0, scaled, 0.0), axis=1, keepdims=True)
    w1 = jnp.sum(jnp.where(lanes == i1, scaled, 0.0), axis=1, keepdims=True)
    idx_ref[...] = jnp.concatenate([i0, i1], axis=1)
    w_ref[...] = jnp.concatenate([w0, w1], axis=1)


def _router(xf, wc_pad, wg_pad, sb):
    return pl.pallas_call(
        _router_body,
        grid=(N // TB,),
        in_specs=[
            pl.BlockSpec((TB, H), lambda t: (t, 0)),
            pl.BlockSpec((H, 128), lambda t: (0, 0)),
            pl.BlockSpec((H, 128), lambda t: (0, 0)),
            pl.BlockSpec((8, 128), lambda t: (0, 0)),
        ],
        out_specs=[
            pl.BlockSpec((TB, 2), lambda t: (t, 0)),
            pl.BlockSpec((TB, 2), lambda t: (t, 0)),
        ],
        out_shape=[
            jax.ShapeDtypeStruct((N, 2), jnp.int32),
            jax.ShapeDtypeStruct((N, 2), jnp.float32),
        ],
    )(xf, wc_pad, wg_pad, sb)


def _meta_body(idx_ref, pos_ref, gid_ref):
    MB = 512
    rows = lax.broadcasted_iota(jnp.int32, (MB, MB), 0)
    cols = lax.broadcasted_iota(jnp.int32, (MB, MB), 1)
    tril = jnp.where(rows >= cols, 1.0, 0.0).astype(jnp.float32)
    lanes = lax.broadcasted_iota(jnp.int32, (MB, 128), 1)

    cnt = jnp.zeros((1, 128), jnp.float32)
    for k in range(NK):
        for blk in range(N // MB):
            e = idx_ref[pl.ds(blk * MB, MB), k:k + 1]
            oh = (e == lanes).astype(jnp.float32)
            cnt = cnt + jnp.sum(oh, axis=0, keepdims=True)
    cnti = cnt.astype(jnp.int32)
    padded = ((cnti + TM - 1) // TM) * TM
    r128 = lax.broadcasted_iota(jnp.int32, (128, 128), 0)
    c128 = lax.broadcasted_iota(jnp.int32, (128, 128), 1)
    sut = jnp.where(r128 < c128, 1.0, 0.0).astype(jnp.float32)
    off = jnp.dot(padded.astype(jnp.float32), sut,
                  preferred_element_type=jnp.float32)

    run = jnp.zeros((1, 128), jnp.float32)
    for k in range(NK):
        for blk in range(N // MB):
            e = idx_ref[pl.ds(blk * MB, MB), k:k + 1]
            oh = (e == lanes).astype(jnp.float32)
            csum = jnp.dot(tril, oh, preferred_element_type=jnp.float32) + run
            rank = jnp.sum(csum * oh, axis=1, keepdims=True) - 1.0
            base = jnp.sum(oh * off, axis=1, keepdims=True)
            pos_ref[pl.ds(blk * MB, MB), k:k + 1] = (rank + base).astype(jnp.int32)
            run = csum[MB - 1:MB, :]

    offi = off.astype(jnp.int32)
    mcol = lax.broadcasted_iota(jnp.int32, (64, 1), 0) * TM
    lanes64 = lax.broadcasted_iota(jnp.int32, (64, 128), 1)
    lm = jnp.logical_and(lanes64 >= 1, lanes64 < E)
    a = jnp.logical_and(mcol >= offi, lm).astype(jnp.int32)
    gid = jnp.sum(a, axis=1, keepdims=True) + (mcol >= CR).astype(jnp.int32)
    gid_ref[...] = gid


def _metadata(idx):
    return pl.pallas_call(
        _meta_body,
        grid=(1,),
        in_specs=[pl.BlockSpec((N, 2), lambda _: (0, 0))],
        out_specs=[
            pl.BlockSpec((N, 2), lambda _: (0, 0)),
            pl.BlockSpec((64, 1), lambda _: (0, 0)),
        ],
        out_shape=[
            jax.ShapeDtypeStruct((N, 2), jnp.int32),
            jax.ShapeDtypeStruct((64, 1), jnp.int32),
        ],
    )(idx)


def _dispatch_body(x_hbm, pos_hbm, wp_hbm, xs_hbm, wsl_hbm,
                   tokv, wv, posv, wpv, rows0, rows1, sem0, sem1):
    core = lax.axis_index("c")
    sub = lax.axis_index("s")
    wid = sub * 2 + core
    pltpu.sync_copy(pos_hbm, posv)
    pltpu.sync_copy(wp_hbm, wpv)

    lane = lax.broadcasted_iota(jnp.int32, (16,), 0)
    zero_i = jnp.zeros((16,), jnp.int32)
    zero_f = jnp.zeros((16,), jnp.float32)

    def init_body(i, _):
        for u in range(4):
            tokv[pl.ds((i * 4 + u) * 16, 16)] = zero_i
            wv[pl.ds((i * 4 + u) * 16, 16)] = zero_f
        return 0

    lax.fori_loop(0, CR // 64, init_body, 0)

    def sc_body(i, _):
        for u in range(4):
            j = i * 4 + u
            pv = posv[pl.ds(j * 16, 16)]
            tv = lax.shift_right_logical(j * 16 + lane, 1)
            wvv = wpv[pl.ds(j * 16, 16)]
            plsc.store_scatter(tokv, [pv], tv)
            plsc.store_scatter(wv, [pv], wvv)
        return 0

    lax.fori_loop(0, (N * NK) // 64, sc_body, 0)

    base = wid * RR
    pltpu.sync_copy(wv.at[pl.ds(base, RR)], wsl_hbm.at[pl.ds(base, RR)])

    bufs = (rows0, rows1)
    sems = (sem0, sem1)
    cp0 = pltpu.async_copy(x_hbm.at[tokv.at[pl.ds(base, CH)]], rows0, sem0)

    def g_body(c, _):
        for u in range(2):
            cc = c * 2 + u
            b = bufs[u]
            s = sems[u]
            nb = bufs[1 - u]
            ns = sems[1 - u]
            pltpu.make_async_copy(x_hbm.at[tokv.at[pl.ds(base, CH)]], b, s).wait()

            @pl.when(cc + 1 < NCH)
            def _():
                start_n = base + (cc + 1) * CH
                pltpu.async_copy(x_hbm.at[tokv.at[pl.ds(start_n, CH)]], nb, ns)

            pltpu.sync_copy(b, xs_hbm.at[pl.ds(base + cc * CH, CH)])
        return 0

    lax.fori_loop(0, NCH // 2, g_body, 0)


def _dispatch(xf, pos2, wp2):
    mesh = plsc.VectorSubcoreMesh(core_axis_name="c", subcore_axis_name="s")
    f = pl.kernel(
        _dispatch_body,
        out_type=[
            jax.ShapeDtypeStruct((CR, H), jnp.float32),
            jax.ShapeDtypeStruct((CR,), jnp.float32),
        ],
        mesh=mesh,
        scratch_types=[
            pltpu.VMEM((CR,), jnp.int32),
            pltpu.VMEM((CR,), jnp.float32),
            pltpu.VMEM((N * NK,), jnp.int32),
            pltpu.VMEM((N * NK,), jnp.float32),
            pltpu.VMEM((CH, H), jnp.float32),
            pltpu.VMEM((CH, H), jnp.float32),
            pltpu.SemaphoreType.DMA,
            pltpu.SemaphoreType.DMA,
        ],
        compiler_params=pltpu.CompilerParams(needs_layout_passes=False),
    )
    return f(xf, pos2, wp2)


def _gmm_body(gid_ref, xs_ref, xf_ref, wg_ref, wu_ref, wd_ref, wsl_ref, out_ref):
    m = pl.program_id(0)
    is_sh = gid_ref[m] == E
    xb = jnp.where(is_sh, xf_ref[...], xs_ref[...]).astype(jnp.bfloat16)
    g = jnp.dot(xb, wg_ref[0], preferred_element_type=jnp.float32)
    u = jnp.dot(xb, wu_ref[0], preferred_element_type=jnp.float32)
    h = (g * jax.nn.sigmoid(g) * u).astype(jnp.bfloat16)
    part = jnp.dot(h, wd_ref[0], preferred_element_type=jnp.float32)
    scale = jnp.where(is_sh, 1.0, wsl_ref[...])
    out_ref[...] = part * scale


def _gmm(gid1d, xs, xf, wg_all, wu_all, wd_all, wsl2):
    grid_spec = pltpu.PrefetchScalarGridSpec(
        num_scalar_prefetch=1,
        grid=(NBLK,),
        in_specs=[
            pl.BlockSpec((TM, H), lambda m, gid: (jnp.minimum(m, MR - 1), 0)),
            pl.BlockSpec((TM, H), lambda m, gid: (jnp.maximum(m - MR, 0), 0)),
            pl.BlockSpec((1, H, I), lambda m, gid: (gid[m], 0, 0)),
            pl.BlockSpec((1, H, I), lambda m, gid: (gid[m], 0, 0)),
            pl.BlockSpec((1, I, H), lambda m, gid: (gid[m], 0, 0)),
            pl.BlockSpec((TM, 1), lambda m, gid: (jnp.minimum(m, MR - 1), 0)),
        ],
        out_specs=pl.BlockSpec((TM, H), lambda m, gid: (m, 0)),
    )
    return pl.pallas_call(
        _gmm_body,
        grid_spec=grid_spec,
        out_shape=jax.ShapeDtypeStruct((P, H), jnp.float32),
        compiler_params=pltpu.CompilerParams(
            dimension_semantics=("arbitrary",),
            vmem_limit_bytes=116 * 1024 * 1024,
        ),
    )(gid1d, xs, xf, wg_all, wu_all, wd_all, wsl2)


def _combine_body(eo_hbm, p0_hbm, p1_hbm, out_hbm,
                  i0v, i1v, bufa, bufb, bufc, sema, semb, semc):
    core = lax.axis_index("c")
    sub = lax.axis_index("s")
    wid = sub * 2 + core
    tb = wid * TPW
    pltpu.sync_copy(p0_hbm.at[pl.ds(tb, TPW)], i0v)
    pltpu.sync_copy(p1_hbm.at[pl.ds(tb, TPW)], i1v)

    def chunk(c, _):
        s = tb + c * CH2
        cpa = pltpu.async_copy(eo_hbm.at[i0v.at[pl.ds(c * CH2, CH2)]], bufa, sema)
        cpb = pltpu.async_copy(eo_hbm.at[i1v.at[pl.ds(c * CH2, CH2)]], bufb, semb)
        cpc = pltpu.async_copy(eo_hbm.at[pl.ds(CR + s, CH2)], bufc, semc)
        cpa.wait()
        cpb.wait()
        cpc.wait()

        def row(r, _):
            def seg(cb, _):
                d = pl.ds(cb * 16, 16)
                bufa[r, d] = bufa[r, d] + bufb[r, d] + bufc[r, d]
                return 0
            lax.fori_loop(0, H // 16, seg, 0)
            return 0

        lax.fori_loop(0, CH2, row, 0)
        pltpu.sync_copy(bufa, out_hbm.at[pl.ds(s, CH2)])
        return 0

    lax.fori_loop(0, TPW // CH2, chunk, 0)


def _combine(eo, p0, p1):
    mesh = plsc.VectorSubcoreMesh(core_axis_name="c", subcore_axis_name="s")
    f = pl.kernel(
        _combine_body,
        out_type=jax.ShapeDtypeStruct((N, H), jnp.float32),
        mesh=mesh,
        scratch_types=[
            pltpu.VMEM((TPW,), jnp.int32),
            pltpu.VMEM((TPW,), jnp.int32),
            pltpu.VMEM((CH2, H), jnp.float32),
            pltpu.VMEM((CH2, H), jnp.float32),
            pltpu.VMEM((CH2, H), jnp.float32),
            pltpu.SemaphoreType.DMA,
            pltpu.SemaphoreType.DMA,
            pltpu.SemaphoreType.DMA,
        ],
        compiler_params=pltpu.CompilerParams(needs_layout_passes=False),
    )
    return f(eo, p0, p1)


def kernel(x, router_gate_W, router_cls_W, extra_scale, extra_bias,
           expert_Wg, expert_Wu, expert_Wd, shared_Wg, shared_Wu, shared_Wd):
    xf = x.reshape(-1, H)
    wc_pad = jnp.zeros((H, 128), jnp.float32).at[:, :E].set(router_cls_W)
    wg_pad = jnp.zeros((H, 128), jnp.float32).at[:, :E].set(router_gate_W)
    sb = (jnp.zeros((8, 128), jnp.float32)
          .at[0, :E].set(extra_scale)
          .at[1, :E].set(extra_bias))
    idx, w = _router(xf, wc_pad, wg_pad, sb)
    pos, gid = _metadata(idx)
    gid1d = gid[:, 0]
    pos2 = pos.reshape(-1)
    wp2 = w.reshape(-1)
    xs, wsl = _dispatch(xf, pos2, wp2)
    wg_all = jnp.concatenate([expert_Wg, shared_Wg[None]], axis=0).astype(jnp.bfloat16)
    wu_all = jnp.concatenate([expert_Wu, shared_Wu[None]], axis=0).astype(jnp.bfloat16)
    wd_all = jnp.concatenate([expert_Wd, shared_Wd[None]], axis=0).astype(jnp.bfloat16)
    eo = _gmm(gid1d, xs, xf, wg_all, wu_all, wd_all, wsl.reshape(CR, 1))
    out = _combine(eo, pos[:, 0], pos[:, 1])
    return out.reshape(x.shape)


# speedup vs baseline: 1.9351x; 1.0941x over previous
import functools

import jax
import jax.numpy as jnp
from jax import lax
from jax.experimental import pallas as pl
from jax.experimental.pallas import tpu as pltpu
from jax.experimental.pallas import tpu_sc as plsc

H = 2048
I = 1408
E = 7
NK = 2
N = 4096
TM = 256
INB = 128
NI = I // INB
CR = 10240
P = CR + N
NBLK = P // TM
MR = CR // TM
TB = 512
NW = 32
RR = CR // NW
TPW = N // NW
CH = 16
NCH = RR // CH
CH2 = 16


def _router_body(x_ref, wc_ref, wg_ref, sb_ref, idx_ref, w_ref):
    xb = x_ref[...]
    c = jnp.dot(xb, wc_ref[...], preferred_element_type=jnp.float32)
    g = jnp.dot(xb, wg_ref[...], preferred_element_type=jnp.float32)
    s = jnp.abs(c * (g * jax.nn.sigmoid(g)))
    lanes = lax.broadcasted_iota(jnp.int32, (TB, 128), 1)
    valid = lanes < E
    neg = jnp.float32(-jnp.inf)
    s = jnp.where(valid, s, neg)
    mx = jnp.max(s, axis=1, keepdims=True)
    ex = jnp.where(valid, jnp.exp(s - mx), 0.0)
    sm = ex / jnp.sum(ex, axis=1, keepdims=True)
    scale_row = sb_ref[0:1, :]
    bias_row = sb_ref[1:2, :]
    biased = jnp.where(valid, sm + bias_row, neg)
    v0 = jnp.max(biased, axis=1, keepdims=True)
    i0 = jnp.min(jnp.where(biased == v0, lanes, 128), axis=1, keepdims=True)
    b2 = jnp.where(lanes == i0, neg, biased)
    v1 = jnp.max(b2, axis=1, keepdims=True)
    i1 = jnp.min(jnp.where(b2 == v1, lanes, 128), axis=1, keepdims=True)
    scaled = 1.0 + sm * scale_row
    w0 = jnp.sum(jnp.where(lanes == i0, scaled, 0.0), axis=1, keepdims=True)
    w1 = jnp.sum(jnp.where(lanes == i1, scaled, 0.0), axis=1, keepdims=True)
    idx_ref[...] = jnp.concatenate([i0, i1], axis=1)
    w_ref[...] = jnp.concatenate([w0, w1], axis=1)


def _router(xf, wc_pad, wg_pad, sb):
    return pl.pallas_call(
        _router_body,
        grid=(N // TB,),
        in_specs=[
            pl.BlockSpec((TB, H), lambda t: (t, 0)),
            pl.BlockSpec((H, 128), lambda t: (0, 0)),
            pl.BlockSpec((H, 128), lambda t: (0, 0)),
            pl.BlockSpec((8, 128), lambda t: (0, 0)),
        ],
        out_specs=[
            pl.BlockSpec((TB, 2), lambda t: (t, 0)),
            pl.BlockSpec((TB, 2), lambda t: (t, 0)),
        ],
        out_shape=[
            jax.ShapeDtypeStruct((N, 2), jnp.int32),
            jax.ShapeDtypeStruct((N, 2), jnp.float32),
        ],
    )(xf, wc_pad, wg_pad, sb)


def _meta_body(idx_ref, pos_ref, gid_ref):
    MB = 512
    rows = lax.broadcasted_iota(jnp.int32, (MB, MB), 0)
    cols = lax.broadcasted_iota(jnp.int32, (MB, MB), 1)
    tril = jnp.where(rows >= cols, 1.0, 0.0).astype(jnp.float32)
    lanes = lax.broadcasted_iota(jnp.int32, (MB, 128), 1)

    cnt = jnp.zeros((1, 128), jnp.float32)
    for k in range(NK):
        for blk in range(N // MB):
            e = idx_ref[pl.ds(blk * MB, MB), k:k + 1]
            oh = (e == lanes).astype(jnp.float32)
            cnt = cnt + jnp.sum(oh, axis=0, keepdims=True)
    cnti = cnt.astype(jnp.int32)
    padded = ((cnti + TM - 1) // TM) * TM
    r128 = lax.broadcasted_iota(jnp.int32, (128, 128), 0)
    c128 = lax.broadcasted_iota(jnp.int32, (128, 128), 1)
    sut = jnp.where(r128 < c128, 1.0, 0.0).astype(jnp.float32)
    off = jnp.dot(padded.astype(jnp.float32), sut,
                  preferred_element_type=jnp.float32)

    run = jnp.zeros((1, 128), jnp.float32)
    for k in range(NK):
        for blk in range(N // MB):
            e = idx_ref[pl.ds(blk * MB, MB), k:k + 1]
            oh = (e == lanes).astype(jnp.float32)
            csum = jnp.dot(tril, oh, preferred_element_type=jnp.float32) + run
            rank = jnp.sum(csum * oh, axis=1, keepdims=True) - 1.0
            base = jnp.sum(oh * off, axis=1, keepdims=True)
            pos_ref[pl.ds(blk * MB, MB), k:k + 1] = (rank + base).astype(jnp.int32)
            run = csum[MB - 1:MB, :]

    offi = off.astype(jnp.int32)
    mcol = lax.broadcasted_iota(jnp.int32, (64, 1), 0) * TM
    lanes64 = lax.broadcasted_iota(jnp.int32, (64, 128), 1)
    lm = jnp.logical_and(lanes64 >= 1, lanes64 < E)
    a = jnp.logical_and(mcol >= offi, lm).astype(jnp.int32)
    gid = jnp.sum(a, axis=1, keepdims=True) + (mcol >= CR).astype(jnp.int32)
    gid_ref[...] = gid


def _metadata(idx):
    return pl.pallas_call(
        _meta_body,
        grid=(1,),
        in_specs=[pl.BlockSpec((N, 2), lambda _: (0, 0))],
        out_specs=[
            pl.BlockSpec((N, 2), lambda _: (0, 0)),
            pl.BlockSpec((64, 1), lambda _: (0, 0)),
        ],
        out_shape=[
            jax.ShapeDtypeStruct((N, 2), jnp.int32),
            jax.ShapeDtypeStruct((64, 1), jnp.int32),
        ],
    )(idx)


def _dispatch_body(x_hbm, pos_hbm, wp_hbm, xs_hbm, wsl_hbm,
                   tokv, wv, posv, wpv, rows0, rows1, sem0, sem1):
    core = lax.axis_index("c")
    sub = lax.axis_index("s")
    wid = sub * 2 + core
    pltpu.sync_copy(pos_hbm, posv)
    pltpu.sync_copy(wp_hbm, wpv)

    lane = lax.broadcasted_iota(jnp.int32, (16,), 0)
    zero_i = jnp.zeros((16,), jnp.int32)
    zero_f = jnp.zeros((16,), jnp.float32)

    def init_body(i, _):
        for u in range(4):
            tokv[pl.ds((i * 4 + u) * 16, 16)] = zero_i
            wv[pl.ds((i * 4 + u) * 16, 16)] = zero_f
        return 0

    lax.fori_loop(0, CR // 64, init_body, 0)

    def sc_body(i, _):
        for u in range(4):
            j = i * 4 + u
            pv = posv[pl.ds(j * 16, 16)]
            tv = lax.shift_right_logical(j * 16 + lane, 1)
            wvv = wpv[pl.ds(j * 16, 16)]
            plsc.store_scatter(tokv, [pv], tv)
            plsc.store_scatter(wv, [pv], wvv)
        return 0

    lax.fori_loop(0, (N * NK) // 64, sc_body, 0)

    base = wid * RR
    pltpu.sync_copy(wv.at[pl.ds(base, RR)], wsl_hbm.at[pl.ds(base, RR)])

    bufs = (rows0, rows1)
    sems = (sem0, sem1)
    cp0 = pltpu.async_copy(x_hbm.at[tokv.at[pl.ds(base, CH)]], rows0, sem0)

    def g_body(c, _):
        for u in range(2):
            cc = c * 2 + u
            b = bufs[u]
            s = sems[u]
            nb = bufs[1 - u]
            ns = sems[1 - u]
            pltpu.make_async_copy(x_hbm.at[tokv.at[pl.ds(base, CH)]], b, s).wait()

            @pl.when(cc + 1 < NCH)
            def _():
                start_n = base + (cc + 1) * CH
                pltpu.async_copy(x_hbm.at[tokv.at[pl.ds(start_n, CH)]], nb, ns)

            pltpu.sync_copy(b, xs_hbm.at[pl.ds(base + cc * CH, CH)])
        return 0

    lax.fori_loop(0, NCH // 2, g_body, 0)


def _dispatch(xf, pos2, wp2):
    mesh = plsc.VectorSubcoreMesh(core_axis_name="c", subcore_axis_name="s")
    f = pl.kernel(
        _dispatch_body,
        out_type=[
            jax.ShapeDtypeStruct((CR, H), jnp.float32),
            jax.ShapeDtypeStruct((CR,), jnp.float32),
        ],
        mesh=mesh,
        scratch_types=[
            pltpu.VMEM((CR,), jnp.int32),
            pltpu.VMEM((CR,), jnp.float32),
            pltpu.VMEM((N * NK,), jnp.int32),
            pltpu.VMEM((N * NK,), jnp.float32),
            pltpu.VMEM((CH, H), jnp.float32),
            pltpu.VMEM((CH, H), jnp.float32),
            pltpu.SemaphoreType.DMA,
            pltpu.SemaphoreType.DMA,
        ],
        compiler_params=pltpu.CompilerParams(needs_layout_passes=False),
    )
    return f(xf, pos2, wp2)


def _stack_cast_body(exp_ref, sh_ref, out_ref):
    e = pl.program_id(0)

    @pl.when(e < E)
    def _():
        out_ref[...] = exp_ref[...].astype(jnp.bfloat16)

    @pl.when(e == E)
    def _():
        out_ref[...] = sh_ref[...].astype(jnp.bfloat16)


def _stack_cast(exp, sh):
    _, d0, d1 = exp.shape
    hb = d0 // 2
    return pl.pallas_call(
        _stack_cast_body,
        grid=(E + 1, 2),
        in_specs=[
            pl.BlockSpec((1, hb, d1), lambda e, h: (jnp.minimum(e, E - 1), h, 0)),
            pl.BlockSpec((1, hb, d1), lambda e, h: (0, jnp.where(e == E, h, 0), 0)),
        ],
        out_specs=pl.BlockSpec((1, hb, d1), lambda e, h: (e, h, 0)),
        out_shape=jax.ShapeDtypeStruct((E + 1, d0, d1), jnp.bfloat16),
        compiler_params=pltpu.CompilerParams(
            dimension_semantics=("arbitrary", "arbitrary"),
        ),
    )(exp, sh)


def _gmm_body(gid_ref, xs_ref, xf_ref, wg_ref, wu_ref, wd_ref, wsl_ref, out_ref):
    m = pl.program_id(0)
    is_sh = gid_ref[m] == E
    xb = jnp.where(is_sh, xf_ref[...], xs_ref[...]).astype(jnp.bfloat16)
    g = jnp.dot(xb, wg_ref[0], preferred_element_type=jnp.float32)
    u = jnp.dot(xb, wu_ref[0], preferred_element_type=jnp.float32)
    h = (g * jax.nn.sigmoid(g) * u).astype(jnp.bfloat16)
    part = jnp.dot(h, wd_ref[0], preferred_element_type=jnp.float32)
    scale = jnp.where(is_sh, 1.0, wsl_ref[...])
    out_ref[...] = part * scale


def _gmm(gid1d, xs, xf, wg_all, wu_all, wd_all, wsl2):
    grid_spec = pltpu.PrefetchScalarGridSpec(
        num_scalar_prefetch=1,
        grid=(NBLK,),
        in_specs=[
            pl.BlockSpec((TM, H), lambda m, gid: (jnp.minimum(m, MR - 1), 0)),
            pl.BlockSpec((TM, H), lambda m, gid: (jnp.maximum(m - MR, 0), 0)),
            pl.BlockSpec((1, H, I), lambda m, gid: (gid[m], 0, 0)),
            pl.BlockSpec((1, H, I), lambda m, gid: (gid[m], 0, 0)),
            pl.BlockSpec((1, I, H), lambda m, gid: (gid[m], 0, 0)),
            pl.BlockSpec((TM, 1), lambda m, gid: (jnp.minimum(m, MR - 1), 0)),
        ],
        out_specs=pl.BlockSpec((TM, H), lambda m, gid: (m, 0)),
    )
    return pl.pallas_call(
        _gmm_body,
        grid_spec=grid_spec,
        out_shape=jax.ShapeDtypeStruct((P, H), jnp.float32),
        compiler_params=pltpu.CompilerParams(
            dimension_semantics=("arbitrary",),
            vmem_limit_bytes=116 * 1024 * 1024,
        ),
    )(gid1d, xs, xf, wg_all, wu_all, wd_all, wsl2)


def _combine_body(eo_hbm, p0_hbm, p1_hbm, out_hbm,
                  i0v, i1v, bufa, bufb, bufc, sema, semb, semc):
    core = lax.axis_index("c")
    sub = lax.axis_index("s")
    wid = sub * 2 + core
    tb = wid * TPW
    pltpu.sync_copy(p0_hbm.at[pl.ds(tb, TPW)], i0v)
    pltpu.sync_copy(p1_hbm.at[pl.ds(tb, TPW)], i1v)

    def chunk(c, _):
        s = tb + c * CH2
        cpa = pltpu.async_copy(eo_hbm.at[i0v.at[pl.ds(c * CH2, CH2)]], bufa, sema)
        cpb = pltpu.async_copy(eo_hbm.at[i1v.at[pl.ds(c * CH2, CH2)]], bufb, semb)
        cpc = pltpu.async_copy(eo_hbm.at[pl.ds(CR + s, CH2)], bufc, semc)
        cpa.wait()
        cpb.wait()
        cpc.wait()

        def row(r, _):
            def seg(cb, _):
                d = pl.ds(cb * 16, 16)
                bufa[r, d] = bufa[r, d] + bufb[r, d] + bufc[r, d]
                return 0
            lax.fori_loop(0, H // 16, seg, 0)
            return 0

        lax.fori_loop(0, CH2, row, 0)
        pltpu.sync_copy(bufa, out_hbm.at[pl.ds(s, CH2)])
        return 0

    lax.fori_loop(0, TPW // CH2, chunk, 0)


def _combine(eo, p0, p1):
    mesh = plsc.VectorSubcoreMesh(core_axis_name="c", subcore_axis_name="s")
    f = pl.kernel(
        _combine_body,
        out_type=jax.ShapeDtypeStruct((N, H), jnp.float32),
        mesh=mesh,
        scratch_types=[
            pltpu.VMEM((TPW,), jnp.int32),
            pltpu.VMEM((TPW,), jnp.int32),
            pltpu.VMEM((CH2, H), jnp.float32),
            pltpu.VMEM((CH2, H), jnp.float32),
            pltpu.VMEM((CH2, H), jnp.float32),
            pltpu.SemaphoreType.DMA,
            pltpu.SemaphoreType.DMA,
            pltpu.SemaphoreType.DMA,
        ],
        compiler_params=pltpu.CompilerParams(needs_layout_passes=False),
    )
    return f(eo, p0, p1)


def kernel(x, router_gate_W, router_cls_W, extra_scale, extra_bias,
           expert_Wg, expert_Wu, expert_Wd, shared_Wg, shared_Wu, shared_Wd):
    xf = x.reshape(-1, H)
    wc_pad = jnp.pad(router_cls_W, ((0, 0), (0, 128 - E)))
    wg_pad = jnp.pad(router_gate_W, ((0, 0), (0, 128 - E)))
    sb = jnp.pad(jnp.stack([extra_scale, extra_bias]),
                 ((0, 6), (0, 128 - E)))
    idx, w = _router(xf, wc_pad, wg_pad, sb)
    pos, gid = _metadata(idx)
    gid1d = gid[:, 0]
    pos2 = pos.reshape(-1)
    wp2 = w.reshape(-1)
    xs, wsl = _dispatch(xf, pos2, wp2)
    wg_all = _stack_cast(expert_Wg, shared_Wg[None])
    wu_all = _stack_cast(expert_Wu, shared_Wu[None])
    wd_all = _stack_cast(expert_Wd, shared_Wd[None])
    eo = _gmm(gid1d, xs, xf, wg_all, wu_all, wd_all, wsl.reshape(CR, 1))
    out = _combine(eo, pos[:, 0], pos[:, 1])
    return out.reshape(x.shape)


# speedup vs baseline: 2.0031x; 1.0351x over previous
import functools

import jax
import jax.numpy as jnp
from jax import lax
from jax.experimental import pallas as pl
from jax.experimental.pallas import tpu as pltpu
from jax.experimental.pallas import tpu_sc as plsc

H = 2048
I = 1408
E = 7
NK = 2
N = 4096
TM = 256
INB = 128
NI = I // INB
CR = 10240
P = CR + N
NBLK = P // TM
MR = CR // TM
TB = 512
NW = 32
RR = CR // NW
TPW = N // NW
CH = 16
NCH = RR // CH
CH2 = 8


def _router_body(x_ref, wc_ref, wg_ref, sb_ref, idx_ref, w_ref):
    xb = x_ref[...]
    c = jnp.dot(xb, wc_ref[...], preferred_element_type=jnp.float32)
    g = jnp.dot(xb, wg_ref[...], preferred_element_type=jnp.float32)
    s = jnp.abs(c * (g * jax.nn.sigmoid(g)))
    lanes = lax.broadcasted_iota(jnp.int32, (TB, 128), 1)
    valid = lanes < E
    neg = jnp.float32(-jnp.inf)
    s = jnp.where(valid, s, neg)
    mx = jnp.max(s, axis=1, keepdims=True)
    ex = jnp.where(valid, jnp.exp(s - mx), 0.0)
    sm = ex / jnp.sum(ex, axis=1, keepdims=True)
    scale_row = sb_ref[0:1, :]
    bias_row = sb_ref[1:2, :]
    biased = jnp.where(valid, sm + bias_row, neg)
    v0 = jnp.max(biased, axis=1, keepdims=True)
    i0 = jnp.min(jnp.where(biased == v0, lanes, 128), axis=1, keepdims=True)
    b2 = jnp.where(lanes == i0, neg, biased)
    v1 = jnp.max(b2, axis=1, keepdims=True)
    i1 = jnp.min(jnp.where(b2 == v1, lanes, 128), axis=1, keepdims=True)
    scaled = 1.0 + sm * scale_row
    w0 = jnp.sum(jnp.where(lanes == i0, scaled, 0.0), axis=1, keepdims=True)
    w1 = jnp.sum(jnp.where(lanes == i1, scaled, 0.0), axis=1, keepdims=True)
    idx_ref[...] = jnp.concatenate([i0, i1], axis=1)
    w_ref[...] = jnp.concatenate([w0, w1], axis=1)


def _router(xf, wc_pad, wg_pad, sb):
    return pl.pallas_call(
        _router_body,
        grid=(N // TB,),
        in_specs=[
            pl.BlockSpec((TB, H), lambda t: (t, 0)),
            pl.BlockSpec((H, 128), lambda t: (0, 0)),
            pl.BlockSpec((H, 128), lambda t: (0, 0)),
            pl.BlockSpec((8, 128), lambda t: (0, 0)),
        ],
        out_specs=[
            pl.BlockSpec((TB, 2), lambda t: (t, 0)),
            pl.BlockSpec((TB, 2), lambda t: (t, 0)),
        ],
        out_shape=[
            jax.ShapeDtypeStruct((N, 2), jnp.int32),
            jax.ShapeDtypeStruct((N, 2), jnp.float32),
        ],
    )(xf, wc_pad, wg_pad, sb)


def _meta_body(idx_ref, pos_ref, gid_ref):
    MB = 512
    rows = lax.broadcasted_iota(jnp.int32, (MB, MB), 0)
    cols = lax.broadcasted_iota(jnp.int32, (MB, MB), 1)
    tril = jnp.where(rows >= cols, 1.0, 0.0).astype(jnp.float32)
    lanes = lax.broadcasted_iota(jnp.int32, (MB, 128), 1)

    cnt = jnp.zeros((1, 128), jnp.float32)
    for k in range(NK):
        for blk in range(N // MB):
            e = idx_ref[pl.ds(blk * MB, MB), k:k + 1]
            oh = (e == lanes).astype(jnp.float32)
            cnt = cnt + jnp.sum(oh, axis=0, keepdims=True)
    cnti = cnt.astype(jnp.int32)
    padded = ((cnti + TM - 1) // TM) * TM
    r128 = lax.broadcasted_iota(jnp.int32, (128, 128), 0)
    c128 = lax.broadcasted_iota(jnp.int32, (128, 128), 1)
    sut = jnp.where(r128 < c128, 1.0, 0.0).astype(jnp.float32)
    off = jnp.dot(padded.astype(jnp.float32), sut,
                  preferred_element_type=jnp.float32)

    run = jnp.zeros((1, 128), jnp.float32)
    for k in range(NK):
        for blk in range(N // MB):
            e = idx_ref[pl.ds(blk * MB, MB), k:k + 1]
            oh = (e == lanes).astype(jnp.float32)
            csum = jnp.dot(tril, oh, preferred_element_type=jnp.float32) + run
            rank = jnp.sum(csum * oh, axis=1, keepdims=True) - 1.0
            base = jnp.sum(oh * off, axis=1, keepdims=True)
            pos_ref[pl.ds(blk * MB, MB), k:k + 1] = (rank + base).astype(jnp.int32)
            run = csum[MB - 1:MB, :]

    offi = off.astype(jnp.int32)
    mcol = lax.broadcasted_iota(jnp.int32, (64, 1), 0) * TM
    lanes64 = lax.broadcasted_iota(jnp.int32, (64, 128), 1)
    lm = jnp.logical_and(lanes64 >= 1, lanes64 < E)
    a = jnp.logical_and(mcol >= offi, lm).astype(jnp.int32)
    gid = jnp.sum(a, axis=1, keepdims=True) + (mcol >= CR).astype(jnp.int32)
    gid_ref[...] = gid


def _metadata(idx):
    return pl.pallas_call(
        _meta_body,
        grid=(1,),
        in_specs=[pl.BlockSpec((N, 2), lambda _: (0, 0))],
        out_specs=[
            pl.BlockSpec((N, 2), lambda _: (0, 0)),
            pl.BlockSpec((64, 1), lambda _: (0, 0)),
        ],
        out_shape=[
            jax.ShapeDtypeStruct((N, 2), jnp.int32),
            jax.ShapeDtypeStruct((64, 1), jnp.int32),
        ],
    )(idx)


def _dispatch_body(x_hbm, pos_hbm, wp_hbm, xs_hbm, wsl_hbm,
                   tokv, wv, posv, wpv, rows0, rows1, sem0, sem1):
    core = lax.axis_index("c")
    sub = lax.axis_index("s")
    wid = sub * 2 + core
    pltpu.sync_copy(pos_hbm, posv)
    pltpu.sync_copy(wp_hbm, wpv)

    lane = lax.broadcasted_iota(jnp.int32, (16,), 0)
    zero_i = jnp.zeros((16,), jnp.int32)
    zero_f = jnp.zeros((16,), jnp.float32)

    def init_body(i, _):
        for u in range(4):
            tokv[pl.ds((i * 4 + u) * 16, 16)] = zero_i
            wv[pl.ds((i * 4 + u) * 16, 16)] = zero_f
        return 0

    lax.fori_loop(0, CR // 64, init_body, 0)

    def sc_body(i, _):
        for u in range(4):
            j = i * 4 + u
            pv = posv[pl.ds(j * 16, 16)]
            tv = lax.shift_right_logical(j * 16 + lane, 1)
            wvv = wpv[pl.ds(j * 16, 16)]
            plsc.store_scatter(tokv, [pv], tv)
            plsc.store_scatter(wv, [pv], wvv)
        return 0

    lax.fori_loop(0, (N * NK) // 64, sc_body, 0)

    base = wid * RR
    pltpu.sync_copy(wv.at[pl.ds(base, RR)], wsl_hbm.at[pl.ds(base, RR)])

    bufs = (rows0, rows1)
    sems = (sem0, sem1)
    cp0 = pltpu.async_copy(x_hbm.at[tokv.at[pl.ds(base, CH)]], rows0, sem0)

    def g_body(c, _):
        for u in range(2):
            cc = c * 2 + u
            b = bufs[u]
            s = sems[u]
            nb = bufs[1 - u]
            ns = sems[1 - u]
            pltpu.make_async_copy(x_hbm.at[tokv.at[pl.ds(base, CH)]], b, s).wait()

            @pl.when(cc + 1 < NCH)
            def _():
                start_n = base + (cc + 1) * CH
                pltpu.async_copy(x_hbm.at[tokv.at[pl.ds(start_n, CH)]], nb, ns)

            pltpu.sync_copy(b, xs_hbm.at[pl.ds(base + cc * CH, CH)])
        return 0

    lax.fori_loop(0, NCH // 2, g_body, 0)


def _dispatch(xf, pos2, wp2):
    mesh = plsc.VectorSubcoreMesh(core_axis_name="c", subcore_axis_name="s")
    f = pl.kernel(
        _dispatch_body,
        out_type=[
            jax.ShapeDtypeStruct((CR, H), jnp.float32),
            jax.ShapeDtypeStruct((CR,), jnp.float32),
        ],
        mesh=mesh,
        scratch_types=[
            pltpu.VMEM((CR,), jnp.int32),
            pltpu.VMEM((CR,), jnp.float32),
            pltpu.VMEM((N * NK,), jnp.int32),
            pltpu.VMEM((N * NK,), jnp.float32),
            pltpu.VMEM((CH, H), jnp.float32),
            pltpu.VMEM((CH, H), jnp.float32),
            pltpu.SemaphoreType.DMA,
            pltpu.SemaphoreType.DMA,
        ],
        compiler_params=pltpu.CompilerParams(needs_layout_passes=False),
    )
    return f(xf, pos2, wp2)


def _stack_cast_body(exp_ref, sh_ref, out_ref):
    e = pl.program_id(0)

    @pl.when(e < E)
    def _():
        out_ref[...] = exp_ref[...].astype(jnp.bfloat16)

    @pl.when(e == E)
    def _():
        out_ref[...] = sh_ref[...].astype(jnp.bfloat16)


def _stack_cast(exp, sh):
    _, d0, d1 = exp.shape
    hb = d0 // 2
    return pl.pallas_call(
        _stack_cast_body,
        grid=(E + 1, 2),
        in_specs=[
            pl.BlockSpec((1, hb, d1), lambda e, h: (jnp.minimum(e, E - 1), h, 0)),
            pl.BlockSpec((1, hb, d1), lambda e, h: (0, jnp.where(e == E, h, 0), 0)),
        ],
        out_specs=pl.BlockSpec((1, hb, d1), lambda e, h: (e, h, 0)),
        out_shape=jax.ShapeDtypeStruct((E + 1, d0, d1), jnp.bfloat16),
        compiler_params=pltpu.CompilerParams(
            dimension_semantics=("arbitrary", "arbitrary"),
        ),
    )(exp, sh)


def _gmm_body(gid_ref, xs_ref, xf_ref, wg_ref, wu_ref, wd_ref, wsl_ref, out_ref):
    m = pl.program_id(0)
    is_sh = gid_ref[m] == E
    xb = jnp.where(is_sh, xf_ref[...], xs_ref[...]).astype(jnp.bfloat16)
    g = jnp.dot(xb, wg_ref[0], preferred_element_type=jnp.float32)
    u = jnp.dot(xb, wu_ref[0], preferred_element_type=jnp.float32)
    h = (g * jax.nn.sigmoid(g) * u).astype(jnp.bfloat16)
    part = jnp.dot(h, wd_ref[0], preferred_element_type=jnp.float32)
    scale = jnp.where(is_sh, 1.0, wsl_ref[...])
    out_ref[...] = part * scale


def _gmm(gid1d, xs, xf, wg_all, wu_all, wd_all, wsl2):
    grid_spec = pltpu.PrefetchScalarGridSpec(
        num_scalar_prefetch=1,
        grid=(NBLK,),
        in_specs=[
            pl.BlockSpec((TM, H), lambda m, gid: (jnp.minimum(m, MR - 1), 0)),
            pl.BlockSpec((TM, H), lambda m, gid: (jnp.maximum(m - MR, 0), 0)),
            pl.BlockSpec((1, H, I), lambda m, gid: (gid[m], 0, 0)),
            pl.BlockSpec((1, H, I), lambda m, gid: (gid[m], 0, 0)),
            pl.BlockSpec((1, I, H), lambda m, gid: (gid[m], 0, 0)),
            pl.BlockSpec((TM, 1), lambda m, gid: (jnp.minimum(m, MR - 1), 0)),
        ],
        out_specs=pl.BlockSpec((TM, H), lambda m, gid: (m, 0)),
    )
    return pl.pallas_call(
        _gmm_body,
        grid_spec=grid_spec,
        out_shape=jax.ShapeDtypeStruct((P, H), jnp.float32),
        compiler_params=pltpu.CompilerParams(
            dimension_semantics=("arbitrary",),
            vmem_limit_bytes=116 * 1024 * 1024,
        ),
    )(gid1d, xs, xf, wg_all, wu_all, wd_all, wsl2)


def _combine_body(eo_hbm, p0_hbm, p1_hbm, out_hbm,
                  i0v, i1v, bufa0, bufb0, bufc0, bufa1, bufb1, bufc1,
                  sem0, sem1):
    core = lax.axis_index("c")
    sub = lax.axis_index("s")
    wid = sub * 2 + core
    tb = wid * TPW
    pltpu.sync_copy(p0_hbm.at[pl.ds(tb, TPW)], i0v)
    pltpu.sync_copy(p1_hbm.at[pl.ds(tb, TPW)], i1v)

    bas = (bufa0, bufa1)
    bbs = (bufb0, bufb1)
    bcs = (bufc0, bufc1)
    sems = (sem0, sem1)
    NCC = TPW // CH2

    def issue(c, k):
        pltpu.async_copy(eo_hbm.at[i0v.at[pl.ds(c * CH2, CH2)]], bas[k], sems[k])
        pltpu.async_copy(eo_hbm.at[i1v.at[pl.ds(c * CH2, CH2)]], bbs[k], sems[k])
        pltpu.async_copy(eo_hbm.at[pl.ds(CR + tb + c * CH2, CH2)], bcs[k], sems[k])

    issue(0, 0)

    def chunk(c, _):
        for k in range(2):
            cc = c * 2 + k
            ba, bb, bc, sm = bas[k], bbs[k], bcs[k], sems[k]
            pltpu.make_async_copy(eo_hbm.at[pl.ds(0, CH2)], ba, sm).wait()
            pltpu.make_async_copy(eo_hbm.at[pl.ds(0, CH2)], bb, sm).wait()
            pltpu.make_async_copy(eo_hbm.at[pl.ds(0, CH2)], bc, sm).wait()

            @pl.when(cc + 1 < NCC)
            def _():
                issue(cc + 1, 1 - k)

            def row(r, _):
                def seg(cb, _):
                    for u in range(4):
                        d = pl.ds((cb * 4 + u) * 16, 16)
                        ba[r, d] = ba[r, d] + bb[r, d] + bc[r, d]
                    return 0
                lax.fori_loop(0, H // 64, seg, 0)
                return 0

            lax.fori_loop(0, CH2, row, 0)
            pltpu.sync_copy(ba, out_hbm.at[pl.ds(tb + cc * CH2, CH2)])
        return 0

    lax.fori_loop(0, NCC // 2, chunk, 0)


def _combine(eo, p0, p1):
    mesh = plsc.VectorSubcoreMesh(core_axis_name="c", subcore_axis_name="s")
    f = pl.kernel(
        _combine_body,
        out_type=jax.ShapeDtypeStruct((N, H), jnp.float32),
        mesh=mesh,
        scratch_types=[
            pltpu.VMEM((TPW,), jnp.int32),
            pltpu.VMEM((TPW,), jnp.int32),
            pltpu.VMEM((CH2, H), jnp.float32),
            pltpu.VMEM((CH2, H), jnp.float32),
            pltpu.VMEM((CH2, H), jnp.float32),
            pltpu.VMEM((CH2, H), jnp.float32),
            pltpu.VMEM((CH2, H), jnp.float32),
            pltpu.VMEM((CH2, H), jnp.float32),
            pltpu.SemaphoreType.DMA,
            pltpu.SemaphoreType.DMA,
        ],
        compiler_params=pltpu.CompilerParams(needs_layout_passes=False),
    )
    return f(eo, p0, p1)


def kernel(x, router_gate_W, router_cls_W, extra_scale, extra_bias,
           expert_Wg, expert_Wu, expert_Wd, shared_Wg, shared_Wu, shared_Wd):
    xf = x.reshape(-1, H)
    wc_pad = jnp.pad(router_cls_W, ((0, 0), (0, 128 - E)))
    wg_pad = jnp.pad(router_gate_W, ((0, 0), (0, 128 - E)))
    sb = jnp.pad(jnp.stack([extra_scale, extra_bias]),
                 ((0, 6), (0, 128 - E)))
    idx, w = _router(xf, wc_pad, wg_pad, sb)
    pos, gid = _metadata(idx)
    gid1d = gid[:, 0]
    pos2 = pos.reshape(-1)
    wp2 = w.reshape(-1)
    xs, wsl = _dispatch(xf, pos2, wp2)
    wg_all = _stack_cast(expert_Wg, shared_Wg[None])
    wu_all = _stack_cast(expert_Wu, shared_Wu[None])
    wd_all = _stack_cast(expert_Wd, shared_Wd[None])
    eo = _gmm(gid1d, xs, xf, wg_all, wu_all, wd_all, wsl.reshape(CR, 1))
    out = _combine(eo, pos[:, 0], pos[:, 1])
    return out.reshape(x.shape)
